# Initial kernel scaffold; baseline (speedup 1.0000x reference)
import jax
import jax.numpy as jnp
from jax.experimental import pallas as pl


def kernel(x, edge_index, batch, W1_0, b1_0, W2_0, b2_0, W1_1, b1_1, W2_1, b2_1, W1_2, b1_2, W2_2, b2_2):
    raise NotImplementedError("write your pallas kernel here")


# speedup vs baseline: 3.9758x; 3.9758x over previous
import functools

import jax
import jax.numpy as jnp
from jax import lax
from jax.experimental import pallas as pl
from jax.experimental.pallas import tpu as pltpu
from jax.experimental.pallas import tpu_sc as plsc

N = 10000
E = 320000
F = 128
D = 128
G = 512

NC = 2
NS = 16
NW = NC * NS

CHUNK = 128
EPT = 10112
E_PAD = EPT * NW
N_PAD = 10240
ROWS_PER_TILE = N_PAD // NS
TRASH_ROW = N

BLK = 400
NBLK = N // BLK


def _sc_agg_body(h_hbm, src_hbm, dst_hbm, zero_hbm, out_hbm,
                 src_idx, dst_idx, rows, zbuf, agg_sh, sem):
    c = lax.axis_index("c")
    s = lax.axis_index("s")
    wid = c * NS + s

    pltpu.sync_copy(zero_hbm, zbuf)
    for j in range(ROWS_PER_TILE // CHUNK):
        pltpu.sync_copy(zbuf, agg_sh.at[pl.ds(s * ROWS_PER_TILE + j * CHUNK, CHUNK)])
    plsc.subcore_barrier()

    ebase = wid * EPT

    def body(j, carry):
        off = ebase + j * CHUNK
        pltpu.sync_copy(src_hbm.at[pl.ds(off, CHUNK)], src_idx)
        pltpu.sync_copy(dst_hbm.at[pl.ds(off, CHUNK)], dst_idx)
        pltpu.async_copy(h_hbm.at[src_idx], rows, sem).wait()
        pltpu.sync_copy(rows, agg_sh.at[dst_idx], add=True)
        return carry

    lax.fori_loop(0, EPT // CHUNK, body, 0)
    plsc.subcore_barrier()

    pltpu.sync_copy(agg_sh.at[pl.ds(s * ROWS_PER_TILE, ROWS_PER_TILE)],
                    out_hbm.at[c].at[pl.ds(s * ROWS_PER_TILE, ROWS_PER_TILE)])


_sc_agg = pl.kernel(
    _sc_agg_body,
    out_type=jax.ShapeDtypeStruct((NC, N_PAD, D), jnp.float32),
    mesh=plsc.VectorSubcoreMesh(core_axis_name="c", subcore_axis_name="s"),
    scratch_types=[
        pltpu.VMEM((CHUNK,), jnp.int32),
        pltpu.VMEM((CHUNK,), jnp.int32),
        pltpu.VMEM((CHUNK, D), jnp.float32),
        pltpu.VMEM((CHUNK, D), jnp.float32),
        pltpu.VMEM_SHARED((N_PAD, D), jnp.float32),
        pltpu.SemaphoreType.DMA,
    ],
)


def _tc_mlp_body(h_ref, agg_ref, batch_ref, w1_ref, b1_ref, w2_ref, b2_ref,
                 hout_ref, pool_ref):
    i = pl.program_id(0)
    z = h_ref[...] + agg_ref[0] + agg_ref[1]
    z1 = jnp.maximum(
        jnp.dot(z, w1_ref[...], preferred_element_type=jnp.float32) + b1_ref[...],
        0.0)
    h2 = jnp.maximum(
        jnp.dot(z1, w2_ref[...], preferred_element_type=jnp.float32) + b2_ref[...],
        0.0)
    hout_ref[...] = h2

    seg = batch_ref[0, 0, :]
    onehot = (seg[None, :] == lax.broadcasted_iota(jnp.int32, (G, BLK), 0)
              ).astype(jnp.float32)
    contrib = jnp.dot(onehot, h2, preferred_element_type=jnp.float32)

    @pl.when(i == 0)
    def _():
        pool_ref[...] = jnp.zeros_like(pool_ref)

    pool_ref[...] += contrib


_tc_mlp = pl.pallas_call(
    _tc_mlp_body,
    grid=(NBLK,),
    in_specs=[
        pl.BlockSpec((BLK, D), lambda i: (i, 0)),
        pl.BlockSpec((NC, BLK, D), lambda i: (0, i, 0)),
        pl.BlockSpec((1, 1, BLK), lambda i: (i, 0, 0)),
        pl.BlockSpec((D, D), lambda i: (0, 0)),
        pl.BlockSpec((1, D), lambda i: (0, 0)),
        pl.BlockSpec((D, D), lambda i: (0, 0)),
        pl.BlockSpec((1, D), lambda i: (0, 0)),
    ],
    out_specs=[
        pl.BlockSpec((BLK, D), lambda i: (i, 0)),
        pl.BlockSpec((G, D), lambda i: (0, 0)),
    ],
    out_shape=[
        jax.ShapeDtypeStruct((N, D), jnp.float32),
        jax.ShapeDtypeStruct((G, D), jnp.float32),
    ],
)


def kernel(x, edge_index, batch, W1_0, b1_0, W2_0, b2_0, W1_1, b1_1, W2_1,
           b2_1, W1_2, b1_2, W2_2, b2_2):
    src = edge_index[0]
    dst = edge_index[1]
    pad = E_PAD - E
    src_p = jnp.concatenate([src, jnp.zeros((pad,), jnp.int32)])
    dst_p = jnp.concatenate([dst, jnp.full((pad,), TRASH_ROW, jnp.int32)])
    zeros128 = jnp.zeros((CHUNK, D), jnp.float32)
    batch3 = batch.reshape(NBLK, 1, BLK)

    params = [
        (W1_0, b1_0.reshape(1, D), W2_0, b2_0.reshape(1, D)),
        (W1_1, b1_1.reshape(1, D), W2_1, b2_1.reshape(1, D)),
        (W1_2, b1_2.reshape(1, D), W2_2, b2_2.reshape(1, D)),
    ]

    h = x
    xs = []
    pools = []
    for (W1, b1, W2, b2) in params:
        agg = _sc_agg(h, src_p, dst_p, zeros128)
        h, pool = _tc_mlp(h, agg, batch3, W1, b1, W2, b2)
        xs.append(h)
        pools.append(pool)

    return (jnp.concatenate(pools, axis=1), jnp.concatenate(xs, axis=1))
